# initial kernel scaffold (unmeasured)
import jax
import jax.numpy as jnp
from jax import lax
from jax.experimental import pallas as pl
from jax.experimental.pallas import tpu as pltpu


def kernel(Q, K, V):
    B, Sq, H, D = Q.shape
    _, Skv, _, _ = K.shape
    G = 4
    HG = H // G
    scale = D ** -0.5

    def body(q_hbm, k_hbm, v_hbm, out_ref,
             q_vmem, k_vmem, v_vmem, load_sems,
             num_send, num_recv, stats_send, stats_recv,
             gbuf, send_sems, recv_sems):
        x = lax.axis_index("x")
        y = lax.axis_index("y")
        z = lax.axis_index("z")
        g = 2 * x + z

        barrier = pltpu.get_barrier_semaphore()
        for nbr in ((1 - x, y, z), (x, 1 - y, z), (x, y, 1 - z)):
            pl.semaphore_signal(barrier, inc=1, device_id=nbr,
                                device_id_type=pl.DeviceIdType.MESH)
        pl.semaphore_wait(barrier, 3)

        cq = pltpu.make_async_copy(
            q_hbm.at[:, :, pl.ds(g * HG, HG), :], q_vmem, load_sems.at[0])
        ck = pltpu.make_async_copy(
            k_hbm.at[:, :, pl.ds(g * HG, HG), :], k_vmem, load_sems.at[1])
        cv = pltpu.make_async_copy(
            v_hbm.at[:, :, pl.ds(g * HG, HG), :], v_vmem, load_sems.at[2])
        cq.start()
        ck.start()
        cv.start()
        cq.wait()
        ck.wait()
        cv.wait()

        for b in range(B):
            for h in range(HG):
                qbh = q_vmem[b, :, h, :].astype(jnp.bfloat16)
                kbh = k_vmem[b, :, h, :].astype(jnp.bfloat16)
                vbh = v_vmem[b, :, h, :].astype(jnp.bfloat16)
                s = lax.dot_general(
                    qbh, kbh, (((1,), (1,)), ((), ())),
                    preferred_element_type=jnp.float32) * scale
                m = jnp.max(s, axis=1)
                p = jnp.exp(s - m[:, None])
                l = jnp.sum(p, axis=1)
                num = lax.dot_general(
                    p.astype(jnp.bfloat16), vbh, (((1,), (0,)), ((), ())),
                    preferred_element_type=jnp.float32)
                num_send[b, :, h, :] = num
                stats_send[0, b, :, h] = m
                stats_send[1, b, :, h] = l

        rn = pltpu.make_async_remote_copy(
            num_send, num_recv, send_sems.at[0], recv_sems.at[0],
            device_id=(x, 1 - y, z), device_id_type=pl.DeviceIdType.MESH)
        rs = pltpu.make_async_remote_copy(
            stats_send, stats_recv, send_sems.at[1], recv_sems.at[1],
            device_id=(x, 1 - y, z), device_id_type=pl.DeviceIdType.MESH)
        rn.start()
        rs.start()
        rn.wait()
        rs.wait()

        m_a = stats_send[0]
        l_a = stats_send[1]
        m_b = stats_recv[0]
        l_b = stats_recv[1]
        m_t = jnp.maximum(m_a, m_b)
        ca = jnp.exp(m_a - m_t)
        cb = jnp.exp(m_b - m_t)
        l_t = l_a * ca + l_b * cb
        num_t = num_send[...] * ca[..., None] + num_recv[...] * cb[..., None]
        out_g = num_t / l_t[..., None]
        gbuf[pl.ds(g, 1)] = out_g[None]

        gz = 2 * x + (1 - z)
        ra = pltpu.make_async_remote_copy(
            gbuf.at[g], gbuf.at[g], send_sems.at[2], recv_sems.at[2],
            device_id=(1 - x, y, z), device_id_type=pl.DeviceIdType.MESH)
        rb = pltpu.make_async_remote_copy(
            gbuf.at[g], gbuf.at[g], send_sems.at[3], recv_sems.at[3],
            device_id=(x, y, 1 - z), device_id_type=pl.DeviceIdType.MESH)
        ra.start()
        rb.start()
        ra.wait()
        rb.wait()

        rc = pltpu.make_async_remote_copy(
            gbuf.at[gz], gbuf.at[gz], send_sems.at[4], recv_sems.at[4],
            device_id=(1 - x, y, z), device_id_type=pl.DeviceIdType.MESH)
        rc.start()
        rc.wait()

        for gg in range(G):
            out_ref[:, :, gg * HG:(gg + 1) * HG, :] = gbuf[gg]

    return pl.pallas_call(
        body,
        out_shape=jax.ShapeDtypeStruct((B, Sq, H, D), jnp.float32),
        in_specs=[
            pl.BlockSpec(memory_space=pltpu.ANY),
            pl.BlockSpec(memory_space=pltpu.ANY),
            pl.BlockSpec(memory_space=pltpu.ANY),
        ],
        out_specs=pl.BlockSpec(memory_space=pltpu.VMEM),
        scratch_shapes=[
            pltpu.VMEM((B, Sq, HG, D), jnp.float32),
            pltpu.VMEM((B, Skv, HG, D), jnp.float32),
            pltpu.VMEM((B, Skv, HG, D), jnp.float32),
            pltpu.SemaphoreType.DMA((3,)),
            pltpu.VMEM((B, Sq, HG, D), jnp.float32),
            pltpu.VMEM((B, Sq, HG, D), jnp.float32),
            pltpu.VMEM((2, B, Sq, HG), jnp.float32),
            pltpu.VMEM((2, B, Sq, HG), jnp.float32),
            pltpu.VMEM((G, B, Sq, HG, D), jnp.float32),
            pltpu.SemaphoreType.DMA((5,)),
            pltpu.SemaphoreType.DMA((5,)),
        ],
        compiler_params=pltpu.CompilerParams(collective_id=0),
    )(Q, K, V)


# baseline (device time: 63148 ns/iter reference)
import jax
import jax.numpy as jnp
from jax import lax
from jax.experimental import pallas as pl
from jax.experimental.pallas import tpu as pltpu


def kernel(Q, K, V):
    B, Sq, H, D = Q.shape
    _, Skv, _, _ = K.shape
    G = 4
    HG = H // G
    scale = D ** -0.5

    def body(q_hbm, k_hbm, v_hbm, out_ref,
             q_vmem, k_vmem, v_vmem, load_sems,
             num_send, num_recv, stats_send, stats_recv,
             gbuf, send_sems, recv_sems):
        x = lax.axis_index("x")
        y = lax.axis_index("y")
        z = lax.axis_index("z")
        g = 2 * x + z

        barrier = pltpu.get_barrier_semaphore()
        for nbr in ((1 - x, y, z), (x, 1 - y, z), (x, y, 1 - z)):
            pl.semaphore_signal(barrier, inc=1, device_id=nbr,
                                device_id_type=pl.DeviceIdType.MESH)
        pl.semaphore_wait(barrier, 3)

        cq = pltpu.make_async_copy(
            q_hbm.at[:, :, pl.ds(g * HG, HG), :], q_vmem, load_sems.at[0])
        ck = pltpu.make_async_copy(
            k_hbm.at[:, :, pl.ds(g * HG, HG), :], k_vmem, load_sems.at[1])
        cv = pltpu.make_async_copy(
            v_hbm.at[:, :, pl.ds(g * HG, HG), :], v_vmem, load_sems.at[2])
        cq.start()
        ck.start()
        cv.start()
        cq.wait()
        ck.wait()
        cv.wait()

        for b in range(B):
            for h in range(HG):
                qbh = q_vmem[b, :, h, :].astype(jnp.bfloat16)
                kbh = k_vmem[b, :, h, :].astype(jnp.bfloat16)
                vbh = v_vmem[b, :, h, :].astype(jnp.bfloat16)
                s = lax.dot_general(
                    qbh, kbh, (((1,), (1,)), ((), ())),
                    preferred_element_type=jnp.float32) * scale
                m = jnp.max(s, axis=1)
                p = jnp.exp(s - m[:, None])
                l = jnp.sum(p, axis=1)
                num = lax.dot_general(
                    p.astype(jnp.bfloat16), vbh, (((1,), (0,)), ((), ())),
                    preferred_element_type=jnp.float32)
                num_send[b, :, h, :] = num
                stats_send[0, b, :, h] = m
                stats_send[1, b, :, h] = l

        rn = pltpu.make_async_remote_copy(
            num_send, num_recv, send_sems.at[0], recv_sems.at[0],
            device_id=(x, 1 - y, z), device_id_type=pl.DeviceIdType.MESH)
        rs = pltpu.make_async_remote_copy(
            stats_send, stats_recv, send_sems.at[1], recv_sems.at[1],
            device_id=(x, 1 - y, z), device_id_type=pl.DeviceIdType.MESH)
        rn.start()
        rs.start()
        rn.wait()
        rs.wait()

        m_a = stats_send[0]
        l_a = stats_send[1]
        m_b = stats_recv[0]
        l_b = stats_recv[1]
        m_t = jnp.maximum(m_a, m_b)
        ca = jnp.exp(m_a - m_t)
        cb = jnp.exp(m_b - m_t)
        l_t = l_a * ca + l_b * cb
        num_t = num_send[...] * ca[..., None] + num_recv[...] * cb[..., None]
        out_g = num_t / l_t[..., None]
        gbuf[pl.ds(g, 1)] = out_g[None]

        gz = 2 * x + (1 - z)
        ra = pltpu.make_async_remote_copy(
            gbuf.at[g], gbuf.at[g], send_sems.at[2], recv_sems.at[2],
            device_id=(1 - x, y, z), device_id_type=pl.DeviceIdType.MESH)
        rb = pltpu.make_async_remote_copy(
            gbuf.at[g], gbuf.at[g], send_sems.at[3], recv_sems.at[3],
            device_id=(x, y, 1 - z), device_id_type=pl.DeviceIdType.MESH)
        ra.start()
        rb.start()
        ra.wait()
        rb.wait()

        rc = pltpu.make_async_remote_copy(
            gbuf.at[gz], gbuf.at[gz], send_sems.at[4], recv_sems.at[4],
            device_id=(1 - x, y, z), device_id_type=pl.DeviceIdType.MESH)
        rc.start()
        rc.wait()

        for gg in range(G):
            out_ref[:, :, gg * HG:(gg + 1) * HG, :] = gbuf[gg]

    return pl.pallas_call(
        body,
        out_shape=jax.ShapeDtypeStruct((B, Sq, H, D), jnp.float32),
        in_specs=[
            pl.BlockSpec(memory_space=pltpu.HBM),
            pl.BlockSpec(memory_space=pltpu.HBM),
            pl.BlockSpec(memory_space=pltpu.HBM),
        ],
        out_specs=pl.BlockSpec(memory_space=pltpu.VMEM),
        scratch_shapes=[
            pltpu.VMEM((B, Sq, HG, D), jnp.float32),
            pltpu.VMEM((B, Skv, HG, D), jnp.float32),
            pltpu.VMEM((B, Skv, HG, D), jnp.float32),
            pltpu.SemaphoreType.DMA((3,)),
            pltpu.VMEM((B, Sq, HG, D), jnp.float32),
            pltpu.VMEM((B, Sq, HG, D), jnp.float32),
            pltpu.VMEM((2, B, Sq, HG), jnp.float32),
            pltpu.VMEM((2, B, Sq, HG), jnp.float32),
            pltpu.VMEM((G, B, Sq, HG, D), jnp.float32),
            pltpu.SemaphoreType.DMA((5,)),
            pltpu.SemaphoreType.DMA((5,)),
        ],
        compiler_params=pltpu.CompilerParams(
            collective_id=0, vmem_limit_bytes=100 * 1024 * 1024),
    )(Q, K, V)


# device time: 28493 ns/iter; 2.2163x vs baseline; 2.2163x over previous
import jax
import jax.numpy as jnp
from jax import lax
from jax.experimental import pallas as pl
from jax.experimental.pallas import tpu as pltpu


def kernel(Q, K, V):
    B, Sq, H, D = Q.shape
    _, Skv, _, _ = K.shape
    G = 4
    HG = H // G
    scale = D ** -0.5

    def body(q_hbm, k_hbm, v_hbm, out_ref,
             q_vmem, k_vmem, v_vmem, load_sems,
             num_send, num_recv, stats_send, stats_recv,
             gbuf, send_sems, recv_sems):
        x = lax.axis_index("x")
        y = lax.axis_index("y")
        z = lax.axis_index("z")
        g = 2 * x + z

        barrier = pltpu.get_barrier_semaphore()
        for nbr in ((1 - x, y, z), (x, 1 - y, z), (x, y, 1 - z)):
            pl.semaphore_signal(barrier, inc=1, device_id=nbr,
                                device_id_type=pl.DeviceIdType.MESH)

        copies = []
        for h in range(HG):
            copies.append(pltpu.make_async_copy(
                q_hbm.at[:, :, g * HG + h, :], q_vmem.at[h],
                load_sems.at[h]))
            copies.append(pltpu.make_async_copy(
                k_hbm.at[:, :, g * HG + h, :], k_vmem.at[h],
                load_sems.at[HG + h]))
            copies.append(pltpu.make_async_copy(
                v_hbm.at[:, :, g * HG + h, :], v_vmem.at[h],
                load_sems.at[2 * HG + h]))
        for c in copies:
            c.start()
        for c in copies:
            c.wait()

        BH = HG * B
        qb = q_vmem[...].astype(jnp.bfloat16).reshape(BH, Sq, D)
        kb = k_vmem[...].astype(jnp.bfloat16).reshape(BH, Skv, D)
        vb = v_vmem[...].astype(jnp.bfloat16).reshape(BH, Skv, D)
        s = lax.dot_general(
            qb, kb, (((2,), (2,)), ((0,), (0,))),
            preferred_element_type=jnp.float32) * scale
        m = jnp.max(s, axis=-1)
        p = jnp.exp(s - m[..., None])
        l = jnp.sum(p, axis=-1)
        num = lax.dot_general(
            p.astype(jnp.bfloat16), vb, (((2,), (1,)), ((0,), (0,))),
            preferred_element_type=jnp.float32)
        num_send[...] = num.astype(jnp.bfloat16).reshape(HG, B, Sq, D)
        stats_send[0] = m.reshape(HG, B, Sq)
        stats_send[1] = l.reshape(HG, B, Sq)

        pl.semaphore_wait(barrier, 3)

        rn = pltpu.make_async_remote_copy(
            num_send, num_recv, send_sems.at[0], recv_sems.at[0],
            device_id=(x, 1 - y, z), device_id_type=pl.DeviceIdType.MESH)
        rs = pltpu.make_async_remote_copy(
            stats_send, stats_recv, send_sems.at[1], recv_sems.at[1],
            device_id=(x, 1 - y, z), device_id_type=pl.DeviceIdType.MESH)
        rn.start()
        rs.start()
        rn.wait()
        rs.wait()

        m_a = stats_send[0]
        l_a = stats_send[1]
        m_b = stats_recv[0]
        l_b = stats_recv[1]
        m_t = jnp.maximum(m_a, m_b)
        ca = jnp.exp(m_a - m_t)
        cb = jnp.exp(m_b - m_t)
        l_t = l_a * ca + l_b * cb
        num_t = (num_send[...].astype(jnp.float32) * ca[..., None]
                 + num_recv[...].astype(jnp.float32) * cb[..., None])
        out_g = num_t / l_t[..., None]
        gbuf[pl.ds(g, 1)] = out_g.astype(jnp.bfloat16)[None]

        gz = 2 * x + (1 - z)
        ra = pltpu.make_async_remote_copy(
            gbuf.at[g], gbuf.at[g], send_sems.at[2], recv_sems.at[2],
            device_id=(1 - x, y, z), device_id_type=pl.DeviceIdType.MESH)
        rb = pltpu.make_async_remote_copy(
            gbuf.at[g], gbuf.at[g], send_sems.at[3], recv_sems.at[3],
            device_id=(x, y, 1 - z), device_id_type=pl.DeviceIdType.MESH)
        ra.start()
        rb.start()
        ra.wait()
        rb.wait()

        rc = pltpu.make_async_remote_copy(
            gbuf.at[gz], gbuf.at[gz], send_sems.at[4], recv_sems.at[4],
            device_id=(1 - x, y, z), device_id_type=pl.DeviceIdType.MESH)
        rc.start()
        rc.wait()

        for gg in range(G):
            for h in range(HG):
                out_ref[:, :, gg * HG + h, :] = (
                    gbuf[gg, h].astype(jnp.float32))

    return pl.pallas_call(
        body,
        out_shape=jax.ShapeDtypeStruct((B, Sq, H, D), jnp.float32),
        in_specs=[
            pl.BlockSpec(memory_space=pltpu.HBM),
            pl.BlockSpec(memory_space=pltpu.HBM),
            pl.BlockSpec(memory_space=pltpu.HBM),
        ],
        out_specs=pl.BlockSpec(memory_space=pltpu.VMEM),
        scratch_shapes=[
            pltpu.VMEM((HG, B, Sq, D), jnp.float32),
            pltpu.VMEM((HG, B, Skv, D), jnp.float32),
            pltpu.VMEM((HG, B, Skv, D), jnp.float32),
            pltpu.SemaphoreType.DMA((3 * HG,)),
            pltpu.VMEM((HG, B, Sq, D), jnp.bfloat16),
            pltpu.VMEM((HG, B, Sq, D), jnp.bfloat16),
            pltpu.VMEM((2, HG, B, Sq), jnp.float32),
            pltpu.VMEM((2, HG, B, Sq), jnp.float32),
            pltpu.VMEM((G, HG, B, Sq, D), jnp.bfloat16),
            pltpu.SemaphoreType.DMA((5,)),
            pltpu.SemaphoreType.DMA((5,)),
        ],
        compiler_params=pltpu.CompilerParams(
            collective_id=0, vmem_limit_bytes=100 * 1024 * 1024),
    )(Q, K, V)


# device time: 25896 ns/iter; 2.4385x vs baseline; 1.1003x over previous
import jax
import jax.numpy as jnp
from jax import lax
from jax.experimental import pallas as pl
from jax.experimental.pallas import tpu as pltpu


def kernel(Q, K, V):
    B, Sq, H, D = Q.shape
    _, Skv, _, _ = K.shape
    G = 4
    HG = H // G
    scale = D ** -0.5

    def body(q_hbm, k_hbm, v_hbm, out_ref,
             q_vmem, k_vmem, v_vmem, load_sems,
             num_send, num_recv, stats_send, stats_recv,
             gbuf, ysend_sems, yrecv_sems, gsend_sems, grecv_sems):
        x = lax.axis_index("x")
        y = lax.axis_index("y")
        z = lax.axis_index("z")
        g = 2 * x + z

        barrier = pltpu.get_barrier_semaphore()
        peers = ((1 - x, y, z), (x, 1 - y, z), (x, y, 1 - z),
                 (1 - x, y, 1 - z))
        for nbr in peers:
            pl.semaphore_signal(barrier, inc=1, device_id=nbr,
                                device_id_type=pl.DeviceIdType.MESH)

        def start_load(h):
            ck = pltpu.make_async_copy(
                k_hbm.at[:, :, g * HG + h, :], k_vmem.at[h],
                load_sems.at[h])
            cv = pltpu.make_async_copy(
                v_hbm.at[:, :, g * HG + h, :], v_vmem.at[h],
                load_sems.at[HG + h])
            ck.start()
            cv.start()
            return ck, cv

        qcopies = [
            pltpu.make_async_copy(
                q_hbm.at[:, :, g * HG + h, :], q_vmem.at[h],
                load_sems.at[2 * HG + h])
            for h in range(HG)
        ]
        for c in qcopies:
            c.start()
        loads = [start_load(0)]

        def make_y_rdmas(h):
            rn = pltpu.make_async_remote_copy(
                num_send.at[h], num_recv.at[h],
                ysend_sems.at[h, 0], yrecv_sems.at[h, 0],
                device_id=(x, 1 - y, z),
                device_id_type=pl.DeviceIdType.MESH)
            rs = pltpu.make_async_remote_copy(
                stats_send.at[h], stats_recv.at[h],
                ysend_sems.at[h, 1], yrecv_sems.at[h, 1],
                device_id=(x, 1 - y, z),
                device_id_type=pl.DeviceIdType.MESH)
            return rn, rs

        def make_gather_rdmas(h):
            dests = ((1 - x, y, z), (x, y, 1 - z), (1 - x, y, 1 - z))
            return [
                pltpu.make_async_remote_copy(
                    gbuf.at[g, h], gbuf.at[g, h],
                    gsend_sems.at[h, j], grecv_sems.at[h, j],
                    device_id=dst, device_id_type=pl.DeviceIdType.MESH)
                for j, dst in enumerate(dests)
            ]

        def compute(h):
            qb = q_vmem[h].astype(jnp.bfloat16)
            kb = k_vmem[h].astype(jnp.bfloat16)
            vb = v_vmem[h].astype(jnp.bfloat16)
            s = lax.dot_general(
                qb, kb, (((2,), (2,)), ((0,), (0,))),
                preferred_element_type=jnp.float32) * scale
            m = jnp.max(s, axis=-1)
            p = jnp.exp(s - m[..., None])
            l = jnp.sum(p, axis=-1)
            num = lax.dot_general(
                p.astype(jnp.bfloat16), vb, (((2,), (1,)), ((0,), (0,))),
                preferred_element_type=jnp.float32)
            num_send[h] = num.astype(jnp.bfloat16)
            stats_send[h, 0] = m
            stats_send[h, 1] = l

        def merge(h):
            m_a = stats_send[h, 0]
            l_a = stats_send[h, 1]
            m_b = stats_recv[h, 0]
            l_b = stats_recv[h, 1]
            m_t = jnp.maximum(m_a, m_b)
            ca = jnp.exp(m_a - m_t)
            cb = jnp.exp(m_b - m_t)
            l_t = l_a * ca + l_b * cb
            num_t = (num_send[h].astype(jnp.float32) * ca[..., None]
                     + num_recv[h].astype(jnp.float32) * cb[..., None])
            out_h = num_t / l_t[..., None]
            gbuf[pl.ds(g, 1), h] = out_h.astype(jnp.bfloat16)[None]

        y_rdmas = [None] * HG
        g_rdmas = [None] * HG
        for h in range(HG):
            loads[h][0].wait()
            loads[h][1].wait()
            if h == 0:
                for c in qcopies:
                    c.wait()
            if h + 1 < HG:
                loads.append(start_load(h + 1))
            compute(h)
            if h == 0:
                pl.semaphore_wait(barrier, len(peers))
            rn, rs = make_y_rdmas(h)
            rn.start()
            rs.start()
            y_rdmas[h] = (rn, rs)
            if h >= 1:
                hp = h - 1
                y_rdmas[hp][0].wait()
                y_rdmas[hp][1].wait()
                merge(hp)
                g_rdmas[hp] = make_gather_rdmas(hp)
                for r in g_rdmas[hp]:
                    r.start()

        y_rdmas[HG - 1][0].wait()
        y_rdmas[HG - 1][1].wait()
        merge(HG - 1)
        g_rdmas[HG - 1] = make_gather_rdmas(HG - 1)
        for r in g_rdmas[HG - 1]:
            r.start()

        gx = 2 * (1 - x) + z
        gz = 2 * x + (1 - z)
        gd = 2 * (1 - x) + (1 - z)
        for h in range(HG):
            for r in g_rdmas[h]:
                r.wait()
            for gg in (g, gx, gz, gd):
                out_ref[:, :, gg * HG + h, :] = gbuf[gg, h].astype(
                    jnp.float32)

    return pl.pallas_call(
        body,
        out_shape=jax.ShapeDtypeStruct((B, Sq, H, D), jnp.float32),
        in_specs=[
            pl.BlockSpec(memory_space=pltpu.HBM),
            pl.BlockSpec(memory_space=pltpu.HBM),
            pl.BlockSpec(memory_space=pltpu.HBM),
        ],
        out_specs=pl.BlockSpec(memory_space=pltpu.VMEM),
        scratch_shapes=[
            pltpu.VMEM((HG, B, Sq, D), jnp.float32),
            pltpu.VMEM((HG, B, Skv, D), jnp.float32),
            pltpu.VMEM((HG, B, Skv, D), jnp.float32),
            pltpu.SemaphoreType.DMA((3 * HG,)),
            pltpu.VMEM((HG, B, Sq, D), jnp.bfloat16),
            pltpu.VMEM((HG, B, Sq, D), jnp.bfloat16),
            pltpu.VMEM((HG, 2, B, Sq), jnp.float32),
            pltpu.VMEM((HG, 2, B, Sq), jnp.float32),
            pltpu.VMEM((G, HG, B, Sq, D), jnp.bfloat16),
            pltpu.SemaphoreType.DMA((HG, 2)),
            pltpu.SemaphoreType.DMA((HG, 2)),
            pltpu.SemaphoreType.DMA((HG, 3)),
            pltpu.SemaphoreType.DMA((HG, 3)),
        ],
        compiler_params=pltpu.CompilerParams(
            collective_id=0, vmem_limit_bytes=100 * 1024 * 1024),
    )(Q, K, V)


# device time: 23244 ns/iter; 2.7167x vs baseline; 1.1141x over previous
import jax
import jax.numpy as jnp
from jax import lax
from jax.experimental import pallas as pl
from jax.experimental.pallas import tpu as pltpu


def kernel(Q, K, V):
    B, Sq, H, D = Q.shape
    _, Skv, _, _ = K.shape
    G = 4
    HG = H // G
    scale = D ** -0.5

    def body(q_hbm, k_hbm, v_hbm, out_ref,
             q_vmem, k_vmem, v_vmem, load_sems,
             num_send, num_recv, stats_send, stats_recv,
             gbuf, ysend_sems, yrecv_sems, gsend_sems, grecv_sems):
        x = lax.axis_index("x")
        y = lax.axis_index("y")
        z = lax.axis_index("z")
        g = 2 * x + z

        barrier = pltpu.get_barrier_semaphore()
        peers = ((1 - x, y, z), (x, 1 - y, z), (x, y, 1 - z),
                 (1 - x, y, 1 - z))
        for nbr in peers:
            pl.semaphore_signal(barrier, inc=1, device_id=nbr,
                                device_id_type=pl.DeviceIdType.MESH)

        def start_load(h):
            ck = pltpu.make_async_copy(
                k_hbm.at[:, :, g * HG + h, :], k_vmem.at[h],
                load_sems.at[h])
            cv = pltpu.make_async_copy(
                v_hbm.at[:, :, g * HG + h, :], v_vmem.at[h],
                load_sems.at[HG + h])
            ck.start()
            cv.start()
            return ck, cv

        qcopies = [
            pltpu.make_async_copy(
                q_hbm.at[:, :, g * HG + h, :], q_vmem.at[h],
                load_sems.at[2 * HG + h])
            for h in range(HG)
        ]
        for c in qcopies:
            c.start()
        loads = [start_load(h) for h in range(HG)]

        def make_y_rdmas(h):
            rn = pltpu.make_async_remote_copy(
                num_send.at[h], num_recv.at[h],
                ysend_sems.at[h, 0], yrecv_sems.at[h, 0],
                device_id=(x, 1 - y, z),
                device_id_type=pl.DeviceIdType.MESH)
            rs = pltpu.make_async_remote_copy(
                stats_send.at[h], stats_recv.at[h],
                ysend_sems.at[h, 1], yrecv_sems.at[h, 1],
                device_id=(x, 1 - y, z),
                device_id_type=pl.DeviceIdType.MESH)
            return rn, rs

        def make_gather_rdmas(h):
            dests = ((1 - x, y, z), (x, y, 1 - z), (1 - x, y, 1 - z))
            return [
                pltpu.make_async_remote_copy(
                    gbuf.at[g, h], gbuf.at[g, h],
                    gsend_sems.at[h, j], grecv_sems.at[h, j],
                    device_id=dst, device_id_type=pl.DeviceIdType.MESH)
                for j, dst in enumerate(dests)
            ]

        def compute(h):
            qb = q_vmem[h].astype(jnp.bfloat16)
            kb = k_vmem[h].astype(jnp.bfloat16)
            vb = v_vmem[h].astype(jnp.bfloat16)
            s = lax.dot_general(
                qb, kb, (((2,), (2,)), ((0,), (0,))),
                preferred_element_type=jnp.float32) * scale
            m = jnp.max(s, axis=-1)
            p = jnp.exp(s - m[..., None])
            l = jnp.sum(p, axis=-1)
            num = lax.dot_general(
                p.astype(jnp.bfloat16), vb, (((2,), (1,)), ((0,), (0,))),
                preferred_element_type=jnp.float32)
            num_send[h] = num.astype(jnp.bfloat16)
            stats_send[h, 0] = m
            stats_send[h, 1] = l

        def merge(h):
            m_a = stats_send[h, 0]
            l_a = stats_send[h, 1]
            m_b = stats_recv[h, 0]
            l_b = stats_recv[h, 1]
            m_t = jnp.maximum(m_a, m_b)
            ca = jnp.exp(m_a - m_t)
            cb = jnp.exp(m_b - m_t)
            l_t = l_a * ca + l_b * cb
            num_t = (num_send[h].astype(jnp.float32) * ca[..., None]
                     + num_recv[h].astype(jnp.float32) * cb[..., None])
            out_h = num_t / l_t[..., None]
            gbuf[pl.ds(g, 1), h] = out_h.astype(jnp.bfloat16)[None]

        y_rdmas = [None] * HG
        g_rdmas = [None] * HG
        for h in range(HG):
            loads[h][0].wait()
            loads[h][1].wait()
            if h == 0:
                for c in qcopies:
                    c.wait()
            compute(h)
            if h == 0:
                pl.semaphore_wait(barrier, len(peers))
            rn, rs = make_y_rdmas(h)
            rn.start()
            rs.start()
            y_rdmas[h] = (rn, rs)
            if h >= 1:
                hp = h - 1
                y_rdmas[hp][0].wait()
                y_rdmas[hp][1].wait()
                merge(hp)
                g_rdmas[hp] = make_gather_rdmas(hp)
                for r in g_rdmas[hp]:
                    r.start()

        y_rdmas[HG - 1][0].wait()
        y_rdmas[HG - 1][1].wait()
        merge(HG - 1)
        g_rdmas[HG - 1] = make_gather_rdmas(HG - 1)
        for r in g_rdmas[HG - 1]:
            r.start()

        gx = 2 * (1 - x) + z
        gz = 2 * x + (1 - z)
        gd = 2 * (1 - x) + (1 - z)
        for h in range(HG):
            for r in g_rdmas[h]:
                r.wait()
            for gg in (g, gx, gz, gd):
                out_ref[:, :, gg * HG + h, :] = gbuf[gg, h].astype(
                    jnp.float32)

    return pl.pallas_call(
        body,
        out_shape=jax.ShapeDtypeStruct((B, Sq, H, D), jnp.float32),
        in_specs=[
            pl.BlockSpec(memory_space=pltpu.HBM),
            pl.BlockSpec(memory_space=pltpu.HBM),
            pl.BlockSpec(memory_space=pltpu.HBM),
        ],
        out_specs=pl.BlockSpec(memory_space=pltpu.VMEM),
        scratch_shapes=[
            pltpu.VMEM((HG, B, Sq, D), jnp.float32),
            pltpu.VMEM((HG, B, Skv, D), jnp.float32),
            pltpu.VMEM((HG, B, Skv, D), jnp.float32),
            pltpu.SemaphoreType.DMA((4 * HG,)),
            pltpu.VMEM((HG, B, Sq, D), jnp.bfloat16),
            pltpu.VMEM((HG, B, Sq, D), jnp.bfloat16),
            pltpu.VMEM((HG, 2, B, Sq), jnp.float32),
            pltpu.VMEM((HG, 2, B, Sq), jnp.float32),
            pltpu.VMEM((G, HG, B, Sq, D), jnp.bfloat16),
            pltpu.SemaphoreType.DMA((HG, 2)),
            pltpu.SemaphoreType.DMA((HG, 2)),
            pltpu.SemaphoreType.DMA((HG, 3)),
            pltpu.SemaphoreType.DMA((HG, 3)),
        ],
        compiler_params=pltpu.CompilerParams(
            collective_id=0, vmem_limit_bytes=100 * 1024 * 1024),
    )(Q, K, V)


# device time: 22989 ns/iter; 2.7469x vs baseline; 1.0111x over previous
import jax
import jax.numpy as jnp
from jax import lax
from jax.experimental import pallas as pl
from jax.experimental.pallas import tpu as pltpu


def kernel(Q, K, V):
    B, Sq, H, D = Q.shape
    _, Skv, _, _ = K.shape
    G = 4
    HG = H // G
    scale = D ** -0.5

    def body(q_hbm, k_hbm, v_hbm, out_ref,
             q_vmem, k_vmem, v_vmem, load_sems,
             pnum, pstats, nsend_sems, ssend_sems, nrecv_sems, srecv_sems):
        x = lax.axis_index("x")
        y = lax.axis_index("y")
        z = lax.axis_index("z")
        g = 2 * x + z

        barrier = pltpu.get_barrier_semaphore()
        peers = ((1 - x, y, z), (x, 1 - y, z), (x, y, 1 - z),
                 (1 - x, 1 - y, z), (1 - x, y, 1 - z), (x, 1 - y, 1 - z),
                 (1 - x, 1 - y, 1 - z))
        for nbr in peers:
            pl.semaphore_signal(barrier, inc=1, device_id=nbr,
                                device_id_type=pl.DeviceIdType.MESH)

        loads = []
        for h in range(HG):
            loads.append((
                pltpu.make_async_copy(
                    k_hbm.at[:, :, g * HG + h, :], k_vmem.at[h],
                    load_sems.at[h]),
                pltpu.make_async_copy(
                    v_hbm.at[:, :, g * HG + h, :], v_vmem.at[h],
                    load_sems.at[HG + h]),
                pltpu.make_async_copy(
                    q_hbm.at[:, :, g * HG + h, :], q_vmem.at[h],
                    load_sems.at[2 * HG + h]),
            ))
        for ld in loads:
            for c in ld:
                c.start()

        def compute(h):
            qb = q_vmem[h].astype(jnp.bfloat16)
            kb = k_vmem[h].astype(jnp.bfloat16)
            vb = v_vmem[h].astype(jnp.bfloat16)
            s = lax.dot_general(
                qb, kb, (((2,), (2,)), ((0,), (0,))),
                preferred_element_type=jnp.float32) * scale
            m = jnp.max(s, axis=-1)
            p = jnp.exp(s - m[..., None])
            l = jnp.sum(p, axis=-1)
            num = lax.dot_general(
                p.astype(jnp.bfloat16), vb, (((2,), (1,)), ((0,), (0,))),
                preferred_element_type=jnp.float32)
            pnum[y, g, h] = num.astype(jnp.bfloat16)
            pstats[y, g, h, 0] = m
            pstats[y, g, h, 1] = l

        def start_sends(h):
            rdmas = []
            for j, dst in enumerate(peers):
                rn = pltpu.make_async_remote_copy(
                    pnum.at[y, g, h], pnum.at[y, g, h],
                    nsend_sems.at[h, j], nrecv_sems.at[y, g, h],
                    device_id=dst, device_id_type=pl.DeviceIdType.MESH)
                rs = pltpu.make_async_remote_copy(
                    pstats.at[y, g, h], pstats.at[y, g, h],
                    ssend_sems.at[h, j], srecv_sems.at[y, g, h],
                    device_id=dst, device_id_type=pl.DeviceIdType.MESH)
                rn.start()
                rs.start()
                rdmas.append((rn, rs))
            return rdmas

        def drain(h, sends):
            for yy in range(2):
                for gg in range(G):
                    is_self = jnp.logical_and(yy == y, gg == g)
                    rn = pltpu.make_async_remote_copy(
                        pnum.at[yy, gg, h], pnum.at[yy, gg, h],
                        nsend_sems.at[h, 0], nrecv_sems.at[yy, gg, h],
                        device_id=(x, y, z),
                        device_id_type=pl.DeviceIdType.MESH)
                    rs = pltpu.make_async_remote_copy(
                        pstats.at[yy, gg, h], pstats.at[yy, gg, h],
                        ssend_sems.at[h, 0], srecv_sems.at[yy, gg, h],
                        device_id=(x, y, z),
                        device_id_type=pl.DeviceIdType.MESH)

                    @pl.when(jnp.logical_not(is_self))
                    def _():
                        rn.wait_recv()
                        rs.wait_recv()

            for rn, rs in sends:
                rn.wait_send()
                rs.wait_send()

            mm = pstats[:, :, h, 0]
            ll = pstats[:, :, h, 1]
            m_t = jnp.maximum(mm[0], mm[1])
            c0 = jnp.exp(mm[0] - m_t)
            c1 = jnp.exp(mm[1] - m_t)
            l_t = ll[0] * c0 + ll[1] * c1
            nums = pnum[:, :, h].astype(jnp.float32)
            num_t = nums[0] * c0[..., None] + nums[1] * c1[..., None]
            out_h = num_t / l_t[..., None]
            for gg in range(G):
                out_ref[:, :, gg * HG + h, :] = out_h[gg]

        sends = [None] * HG
        for h in range(HG):
            for c in loads[h]:
                c.wait()
            compute(h)
            if h == 0:
                pl.semaphore_wait(barrier, len(peers))
            sends[h] = start_sends(h)
            if h >= 2:
                drain(h - 2, sends[h - 2])
        drain(HG - 2, sends[HG - 2])
        drain(HG - 1, sends[HG - 1])

    return pl.pallas_call(
        body,
        out_shape=jax.ShapeDtypeStruct((B, Sq, H, D), jnp.float32),
        in_specs=[
            pl.BlockSpec(memory_space=pltpu.HBM),
            pl.BlockSpec(memory_space=pltpu.HBM),
            pl.BlockSpec(memory_space=pltpu.HBM),
        ],
        out_specs=pl.BlockSpec(memory_space=pltpu.VMEM),
        scratch_shapes=[
            pltpu.VMEM((HG, B, Sq, D), jnp.float32),
            pltpu.VMEM((HG, B, Skv, D), jnp.float32),
            pltpu.VMEM((HG, B, Skv, D), jnp.float32),
            pltpu.SemaphoreType.DMA((3 * HG,)),
            pltpu.VMEM((2, G, HG, B, Sq, D), jnp.bfloat16),
            pltpu.VMEM((2, G, HG, 2, B, Sq), jnp.float32),
            pltpu.SemaphoreType.DMA((HG, 7)),
            pltpu.SemaphoreType.DMA((HG, 7)),
            pltpu.SemaphoreType.DMA((2, G, HG)),
            pltpu.SemaphoreType.DMA((2, G, HG)),
        ],
        compiler_params=pltpu.CompilerParams(
            collective_id=0, vmem_limit_bytes=100 * 1024 * 1024),
    )(Q, K, V)
